# baseline (speedup 1.0000x reference)
import functools

import jax
import jax.numpy as jnp
from jax import lax
from jax.experimental import pallas as pl
from jax.experimental.pallas import tpu as pltpu
from jax.experimental.pallas import tpu_sc as plsc

N_NODES = 100000
N_EDGES = 3200000
N_PAD = 100096

NC, NS, L = 2, 16, 16
NW = NC * NS
NTILES = N_EDGES // 128
CT = 11
W = CT * 128
NFULL = 71


def _pack_body(xt_ref, we_ref, be_ref, out_ref):
    xt = xt_ref[...]
    we = we_ref[...]
    be = be_ref[...]
    s_src = (xt[0] * we[0, 0] + xt[1] * we[1, 0]
             + xt[2] * we[2, 0] + xt[3] * we[3, 0] + be[0, 0])
    s_dst = (xt[0] * we[4, 0] + xt[1] * we[5, 0]
             + xt[2] * we[6, 0] + xt[3] * we[7, 0])
    lo = lax.bitcast_convert_type(s_src.astype(jnp.bfloat16), jnp.uint16)
    hi = lax.bitcast_convert_type(s_dst.astype(jnp.bfloat16), jnp.uint16)
    packed = lo.astype(jnp.uint32) | (hi.astype(jnp.uint32) << 16)
    out_ref[...] = lax.bitcast_convert_type(packed, jnp.int32)[None]


_pack = pl.pallas_call(
    _pack_body,
    out_shape=jax.ShapeDtypeStruct((1, N_PAD), jnp.int32),
)


@functools.partial(
    pl.kernel,
    out_type=jax.ShapeDtypeStruct((N_EDGES,), jnp.float32),
    mesh=plsc.VectorSubcoreMesh(core_axis_name="c", subcore_axis_name="s"),
    compiler_params=pltpu.CompilerParams(needs_layout_passes=False),
    scratch_types=(
        [pltpu.VMEM((N_PAD,), jnp.int32)]
        + [pltpu.VMEM((2, W), jnp.int32)] * 2
        + [pltpu.VMEM((W,), jnp.float32)] * 4
        + [pltpu.VMEM((L,), jnp.float32)]
        + [pltpu.SemaphoreType.DMA] * 4
    ),
)
def _edge_kernel(packed_hbm, ei_hbm, ea_hbm, w8_hbm, out_hbm,
                 table_v, idx0, idx1, ea0, ea1, o0, o1, w8_v,
                 isem0, isem1, osem0, osem1):
    wid = lax.axis_index("s") * NC + lax.axis_index("c")
    tbase = 781 * wid + jnp.minimum(wid, 8)
    bufs = ((idx0, ea0, o0, isem0, osem0),
            (idx1, ea1, o1, isem1, osem1))

    def in_copies(ct, slot, n):
        ib, eb, _, isem, _ = bufs[slot]
        eoff = pl.multiple_of(ct * 128, 128)
        return (
            pltpu.make_async_copy(ei_hbm.at[:, pl.ds(eoff, n * 128)],
                                  ib.at[:, pl.ds(0, n * 128)], isem),
            pltpu.make_async_copy(ea_hbm.at[pl.ds(eoff, n * 128)],
                                  eb.at[pl.ds(0, n * 128)], isem),
        )

    def out_copy(ct, slot, n):
        ob, osem = bufs[slot][2], bufs[slot][4]
        eoff = pl.multiple_of(ct * 128, 128)
        return pltpu.make_async_copy(ob.at[pl.ds(0, n * 128)],
                                     out_hbm.at[pl.ds(eoff, n * 128)], osem)

    def compute(slot, n):
        ib, eb, ob, _, _ = bufs[slot]
        w8 = w8_v[...]

        @plsc.parallel_loop(0, n * 128, step=L, unroll=8)
        def body(i):
            sl = pl.ds(pl.multiple_of(i, L), L)
            wr = plsc.load_gather(table_v, [ib[0, sl]])
            wc = plsc.load_gather(table_v, [ib[1, sl]])
            a = plsc.bitcast(wr << 16, jnp.float32)
            b = plsc.bitcast(wc & jnp.int32(-65536), jnp.float32)
            ob[sl] = jnp.maximum(a + b + w8 * eb[sl], 0.0)

    for c in in_copies(tbase, 0, CT):
        c.start()
    pltpu.sync_copy(packed_hbm, table_v)
    pltpu.sync_copy(w8_hbm, w8_v)

    def process(g, slot):
        ct = tbase + g * CT
        for c in in_copies(ct, slot, CT):
            c.wait()

        @pl.when(g + 1 < NFULL)
        def _():
            for c in in_copies(tbase + (g + 1) * CT, 1 - slot, CT):
                c.start()

        @pl.when(g >= 2)
        def _():
            out_copy(tbase + (g - 2) * CT, slot, CT).wait()

        compute(slot, CT)
        out_copy(ct, slot, CT).start()

    def loop_body(k, carry):
        process(2 * k, 0)
        process(2 * k + 1, 1)
        return carry

    lax.fori_loop(0, NFULL // 2, loop_body, 0)
    process(NFULL - 1, 0)

    @pl.when(wid < 8)
    def _():
        et = tbase + NFULL * CT
        for c in in_copies(et, 1, 1):
            c.start()
        out_copy(tbase + (NFULL - 2) * CT, 1, CT).wait()
        for c in in_copies(et, 1, 1):
            c.wait()
        compute(1, 1)
        out_copy(et, 1, 1).start()
        out_copy(et, 1, 1).wait()

    out_copy(tbase + (NFULL - 1) * CT, 0, CT).wait()

    @pl.when(wid >= 8)
    def _():
        out_copy(tbase + (NFULL - 2) * CT, 1, CT).wait()


def kernel(x, edge_index, edge_attr, W_e, b_e, W_n, b_n):
    xt = jnp.pad(x, ((0, N_PAD - N_NODES), (0, 0))).T
    packed = _pack(xt, W_e, b_e.reshape(1, 1))[0]
    ea = edge_attr.reshape(N_EDGES)
    w8 = jnp.full((L,), W_e[8, 0], dtype=jnp.float32)
    out = _edge_kernel(packed, edge_index, ea, w8)
    return out.reshape(N_EDGES, 1)


# speedup vs baseline: 1.2836x; 1.2836x over previous
import functools

import jax
import jax.numpy as jnp
from jax import lax
from jax.experimental import pallas as pl
from jax.experimental.pallas import tpu as pltpu
from jax.experimental.pallas import tpu_sc as plsc

N_NODES = 100000
N_EDGES = 3200000
N_PAD = 100096

NC, NS, L = 2, 16, 16
NW = NC * NS
EPT = N_EDGES // NW
CHUNK = 2000
NCHUNK = EPT // CHUNK


def _pack_body(xt_ref, we_ref, be_ref, out_ref):
    xt = xt_ref[...]
    we = we_ref[...]
    be = be_ref[...]
    s_src = (xt[0] * we[0, 0] + xt[1] * we[1, 0]
             + xt[2] * we[2, 0] + xt[3] * we[3, 0] + be[0, 0])
    s_dst = (xt[0] * we[4, 0] + xt[1] * we[5, 0]
             + xt[2] * we[6, 0] + xt[3] * we[7, 0])
    lo = lax.bitcast_convert_type(s_src.astype(jnp.bfloat16), jnp.uint16)
    hi = lax.bitcast_convert_type(s_dst.astype(jnp.bfloat16), jnp.uint16)
    packed = lo.astype(jnp.uint32) | (hi.astype(jnp.uint32) << 16)
    out_ref[...] = lax.bitcast_convert_type(packed, jnp.int32)[None]


_pack = pl.pallas_call(
    _pack_body,
    out_shape=jax.ShapeDtypeStruct((1, N_PAD), jnp.int32),
)


@functools.partial(
    pl.kernel,
    out_type=jax.ShapeDtypeStruct((N_EDGES,), jnp.float32),
    mesh=plsc.VectorSubcoreMesh(core_axis_name="c", subcore_axis_name="s"),
    compiler_params=pltpu.CompilerParams(needs_layout_passes=False),
    scratch_types=(
        [pltpu.VMEM((N_PAD,), jnp.int32)]
        + [pltpu.VMEM((CHUNK,), jnp.int32)] * 6
        + [pltpu.VMEM((CHUNK,), jnp.float32)] * 6
        + [pltpu.VMEM((L,), jnp.float32)]
        + [pltpu.SemaphoreType.DMA] * 6
    ),
)
def _edge_kernel(packed_hbm, ei_hbm, ea_hbm, w8_hbm, out_hbm,
                 table_v, row0, row1, row2, col0, col1, col2,
                 ea0, ea1, ea2, o0, o1, o2, w8_v,
                 isem0, isem1, isem2, osem0, osem1, osem2):
    wid = lax.axis_index("s") * NC + lax.axis_index("c")
    base = wid * EPT
    bufs = ((row0, col0, ea0, o0, isem0, osem0),
            (row1, col1, ea1, o1, isem1, osem1),
            (row2, col2, ea2, o2, isem2, osem2))

    def in_copies(g, slot):
        rb, cb, eb, _, isem, _ = bufs[slot]
        off = base + g * CHUNK
        return (
            pltpu.make_async_copy(ei_hbm.at[pl.ds(off, CHUNK)], rb, isem),
            pltpu.make_async_copy(ei_hbm.at[pl.ds(N_EDGES + off, CHUNK)],
                                  cb, isem),
            pltpu.make_async_copy(ea_hbm.at[pl.ds(off, CHUNK)], eb, isem),
        )

    def out_copy(g, slot):
        ob, osem = bufs[slot][3], bufs[slot][5]
        off = base + g * CHUNK
        return pltpu.make_async_copy(ob, out_hbm.at[pl.ds(off, CHUNK)], osem)

    for c in in_copies(0, 0) + in_copies(1, 1):
        c.start()
    pltpu.sync_copy(packed_hbm, table_v)
    pltpu.sync_copy(w8_hbm, w8_v)
    w8 = w8_v[...]

    def process(g, slot):
        rb, cb, eb, ob, _, _ = bufs[slot]
        for c in in_copies(g, slot):
            c.wait()

        @pl.when(g + 2 < NCHUNK)
        def _():
            for c in in_copies(g + 2, (slot + 2) % 3):
                c.start()

        @pl.when(g >= 3)
        def _():
            out_copy(g - 3, slot).wait()

        @plsc.parallel_loop(0, CHUNK, step=L, unroll=8)
        def body(i):
            sl = pl.ds(pl.multiple_of(i, L), L)
            wr = plsc.load_gather(table_v, [rb[sl]])
            wc = plsc.load_gather(table_v, [cb[sl]])
            a = plsc.bitcast(wr << 16, jnp.float32)
            b = plsc.bitcast(wc & jnp.int32(-65536), jnp.float32)
            ob[sl] = jnp.maximum(a + b + w8 * eb[sl], 0.0)

        out_copy(g, slot).start()

    def loop_body(g, carry):
        for k in range(3):
            @pl.when(lax.rem(g, 3) == k)
            def _():
                process(g, k)
        return carry

    lax.fori_loop(0, NCHUNK, loop_body, 0)
    for g in range(NCHUNK - 3, NCHUNK):
        out_copy(g, g % 3).wait()


def kernel(x, edge_index, edge_attr, W_e, b_e, W_n, b_n):
    xt = jnp.pad(x, ((0, N_PAD - N_NODES), (0, 0))).T
    packed = _pack(xt, W_e, b_e.reshape(1, 1))[0]
    ea = edge_attr.reshape(N_EDGES)
    ei = edge_index.reshape(2 * N_EDGES)
    w8 = jnp.full((L,), W_e[8, 0], dtype=jnp.float32)
    out = _edge_kernel(packed, ei, ea, w8)
    return out.reshape(N_EDGES, 1)


# speedup vs baseline: 1.2871x; 1.0027x over previous
import functools

import jax
import jax.numpy as jnp
from jax import lax
from jax.experimental import pallas as pl
from jax.experimental.pallas import tpu as pltpu
from jax.experimental.pallas import tpu_sc as plsc

N_NODES = 100000
N_EDGES = 3200000
N_PAD = 100096

NC, NS, L = 2, 16, 16
NW = NC * NS
EPT = N_EDGES // NW
CHUNK = 2000
NCHUNK = EPT // CHUNK


def _pack_body(xt_ref, we_ref, be_ref, out_ref):
    xt = xt_ref[...]
    we = we_ref[...]
    be = be_ref[...]
    s_src = (xt[0] * we[0, 0] + xt[1] * we[1, 0]
             + xt[2] * we[2, 0] + xt[3] * we[3, 0] + be[0, 0])
    s_dst = (xt[0] * we[4, 0] + xt[1] * we[5, 0]
             + xt[2] * we[6, 0] + xt[3] * we[7, 0])
    lo = lax.bitcast_convert_type(s_src.astype(jnp.bfloat16), jnp.uint16)
    hi = lax.bitcast_convert_type(s_dst.astype(jnp.bfloat16), jnp.uint16)
    packed = lo.astype(jnp.uint32) | (hi.astype(jnp.uint32) << 16)
    out_ref[...] = lax.bitcast_convert_type(packed, jnp.int32)[None]


_pack = pl.pallas_call(
    _pack_body,
    out_shape=jax.ShapeDtypeStruct((1, N_PAD), jnp.int32),
)


@functools.partial(
    pl.kernel,
    out_type=jax.ShapeDtypeStruct((N_EDGES,), jnp.float32),
    mesh=plsc.VectorSubcoreMesh(core_axis_name="c", subcore_axis_name="s"),
    compiler_params=pltpu.CompilerParams(needs_layout_passes=False),
    scratch_types=(
        [pltpu.VMEM((N_PAD,), jnp.int32)]
        + [pltpu.VMEM((CHUNK,), jnp.int32)] * 8
        + [pltpu.VMEM((CHUNK,), jnp.float32)] * 4
        + [pltpu.VMEM((L,), jnp.float32)]
        + [pltpu.SemaphoreType.DMA] * 8
    ),
)
def _edge_kernel(packed_hbm, ei_hbm, ea_hbm, w8_hbm, out_hbm,
                 table_v, row0, row1, row2, row3, col0, col1, col2, col3,
                 eo0, eo1, eo2, eo3, w8_v,
                 isem0, isem1, isem2, isem3, osem0, osem1, osem2, osem3):
    wid = lax.axis_index("s") * NC + lax.axis_index("c")
    base = wid * EPT
    bufs = ((row0, col0, eo0, isem0, osem0),
            (row1, col1, eo1, isem1, osem1),
            (row2, col2, eo2, isem2, osem2),
            (row3, col3, eo3, isem3, osem3))
    NSLOT = 4

    def in_copies(g, slot):
        rb, cb, eb, isem, _ = bufs[slot]
        off = base + g * CHUNK
        return (
            pltpu.make_async_copy(ei_hbm.at[pl.ds(off, CHUNK)], rb, isem),
            pltpu.make_async_copy(ei_hbm.at[pl.ds(N_EDGES + off, CHUNK)],
                                  cb, isem),
            pltpu.make_async_copy(ea_hbm.at[pl.ds(off, CHUNK)], eb, isem),
        )

    def out_copy(g, slot):
        ob, osem = bufs[slot][2], bufs[slot][4]
        off = base + g * CHUNK
        return pltpu.make_async_copy(ob, out_hbm.at[pl.ds(off, CHUNK)], osem)

    for c in in_copies(0, 0) + in_copies(1, 1):
        c.start()
    pltpu.sync_copy(packed_hbm, table_v)
    pltpu.sync_copy(w8_hbm, w8_v)
    w8 = w8_v[...]

    def process(g, slot):
        rb, cb, eb, _, _ = bufs[slot]
        for c in in_copies(g, slot):
            c.wait()

        @pl.when(g >= 2)
        def _():
            out_copy(g - 2, (slot + 2) % NSLOT).wait()

        @pl.when(g + 2 < NCHUNK)
        def _():
            for c in in_copies(g + 2, (slot + 2) % NSLOT):
                c.start()

        @plsc.parallel_loop(0, CHUNK, step=L, unroll=8)
        def body(i):
            sl = pl.ds(pl.multiple_of(i, L), L)
            wr = plsc.load_gather(table_v, [rb[sl]])
            wc = plsc.load_gather(table_v, [cb[sl]])
            a = plsc.bitcast(wr << 16, jnp.float32)
            b = plsc.bitcast(wc & jnp.int32(-65536), jnp.float32)
            eb[sl] = jnp.maximum(a + b + w8 * eb[sl], 0.0)

        out_copy(g, slot).start()

    def loop_body(g, carry):
        for k in range(NSLOT):
            @pl.when(lax.rem(g, NSLOT) == k)
            def _():
                process(g, k)
        return carry

    lax.fori_loop(0, NCHUNK, loop_body, 0)
    for g in range(NCHUNK - 2, NCHUNK):
        out_copy(g, g % NSLOT).wait()


def kernel(x, edge_index, edge_attr, W_e, b_e, W_n, b_n):
    xt = jnp.pad(x, ((0, N_PAD - N_NODES), (0, 0))).T
    packed = _pack(xt, W_e, b_e.reshape(1, 1))[0]
    ea = edge_attr.reshape(N_EDGES)
    ei = edge_index.reshape(2 * N_EDGES)
    w8 = jnp.full((L,), W_e[8, 0], dtype=jnp.float32)
    out = _edge_kernel(packed, ei, ea, w8)
    return out.reshape(N_EDGES, 1)
